# Initial kernel scaffold; baseline (speedup 1.0000x reference)
import jax
import jax.numpy as jnp
from jax.experimental import pallas as pl


def kernel(input_ids, position_ids, W_tok, W_pos, ln_gamma, ln_beta):
    raise NotImplementedError("write your pallas kernel here")


# speedup vs baseline: 1.0762x; 1.0762x over previous
import functools

import jax
import jax.numpy as jnp
from jax import lax
from jax.experimental import pallas as pl
from jax.experimental.pallas import tpu as pltpu
from jax.experimental.pallas import tpu_sc as plsc

NC = 2
NS = 16
LANES = 16
NW = NC * NS

D = 128
DC = D // LANES
IDX_CHUNK = 128


def _make_sc_kernel(N, S):
    rows_pw = N // NW
    n_gather = rows_pw // IDX_CHUNK

    mesh = plsc.VectorSubcoreMesh(
        core_axis_name="c", subcore_axis_name="s",
        num_cores=NC, num_subcores=NS)

    @functools.partial(
        pl.kernel,
        out_type=jax.ShapeDtypeStruct((N, D), jnp.float32),
        mesh=mesh,
        scratch_types=[
            pltpu.VMEM((rows_pw,), jnp.int32),
            pltpu.VMEM((rows_pw,), jnp.int32),
            pltpu.VMEM((rows_pw, D), jnp.float32),
            pltpu.VMEM((rows_pw, D), jnp.float32),
            pltpu.VMEM((D,), jnp.float32),
            pltpu.VMEM((D,), jnp.float32),
            pltpu.SemaphoreType.DMA,
        ],
    )
    def emb_kernel(ids_hbm, pids_hbm, wtok_hbm, wpos_hbm, gam_hbm, bet_hbm,
                   out_hbm, idx_v, pidx_v, tok_v, pos_v, gam_v, bet_v, sem):
        wid = lax.axis_index("s") * NC + lax.axis_index("c")
        base = wid * rows_pw
        sbase = lax.rem(base, S)

        pltpu.sync_copy(ids_hbm.at[pl.ds(base, rows_pw)], idx_v)
        pltpu.sync_copy(pids_hbm.at[pl.ds(sbase, rows_pw)], pidx_v)
        pltpu.sync_copy(gam_hbm, gam_v)
        pltpu.sync_copy(bet_hbm, bet_v)

        copies = []
        for j in range(n_gather):
            sl = pl.ds(j * IDX_CHUNK, IDX_CHUNK)
            copies.append(pltpu.make_async_copy(
                wtok_hbm.at[idx_v.at[sl]], tok_v.at[sl], sem))
            copies.append(pltpu.make_async_copy(
                wpos_hbm.at[pidx_v.at[sl]], pos_v.at[sl], sem))
        for c in copies:
            c.start()
        for c in copies:
            c.wait()

        gam = [gam_v[pl.ds(c * LANES, LANES)] for c in range(DC)]
        bet = [bet_v[pl.ds(c * LANES, LANES)] for c in range(DC)]

        inv_d = 1.0 / D
        lane = lax.iota(jnp.int32, LANES)
        perms = [lane ^ k for k in (1, 2, 4, 8)]

        def _hsum(v):
            for p in perms:
                v = v + v.at[p].get(mode="promise_in_bounds")
            return v

        def row_body(r, carry):
            xs = []
            s1 = jnp.zeros((LANES,), jnp.float32)
            s2 = jnp.zeros((LANES,), jnp.float32)
            for c in range(DC):
                x = (tok_v[r, pl.ds(c * LANES, LANES)]
                     + pos_v[r, pl.ds(c * LANES, LANES)])
                xs.append(x)
                s1 = s1 + x
                s2 = s2 + x * x
            mean = _hsum(s1) * inv_d
            var = _hsum(s2) * inv_d - mean * mean
            a = var + 1e-5
            i = lax.bitcast_convert_type(a, jnp.int32)
            i = 0x5F3759DF - lax.shift_right_logical(i, 1)
            y = lax.bitcast_convert_type(i, jnp.float32)
            half_a = 0.5 * a
            for _ in range(3):
                y = y * (1.5 - half_a * y * y)
            for c in range(DC):
                tok_v[r, pl.ds(c * LANES, LANES)] = (
                    (xs[c] - mean) * y * gam[c] + bet[c])
            return carry

        lax.fori_loop(0, rows_pw, row_body, 0, unroll=2)

        pltpu.sync_copy(tok_v, out_hbm.at[pl.ds(base, rows_pw)])

    return emb_kernel


_SC_KERNEL_CACHE = {}


def kernel(input_ids, position_ids, W_tok, W_pos, ln_gamma, ln_beta):
    B, S = input_ids.shape
    N = B * S
    key = (N, S)
    if key not in _SC_KERNEL_CACHE:
        _SC_KERNEL_CACHE[key] = _make_sc_kernel(N, S)
    ids = input_ids.reshape(N)
    out = _SC_KERNEL_CACHE[key](ids, position_ids, W_tok, W_pos,
                                ln_gamma, ln_beta)
    return out.reshape(B, S, D)
